# baseline (speedup 1.0000x reference)
import jax
import jax.numpy as jnp
from jax import lax
from jax.experimental import pallas as pl
from jax.experimental.pallas import tpu as pltpu
from jax.experimental.pallas import tpu_sc as plsc

NN = 10000
EE = 160000
DD = 256
HALF = 128
NC = 2
NS = 16
EPT = EE // NS
ACH = 80
ANCH = 125
GCH = 128
GNCH = 80
DW = 128
NPAD = 10240
DSTR = NPAD // NS
BM = 2000
GRID = NN // BM


def _sc_mesh():
    return plsc.VectorSubcoreMesh(
        core_axis_name="c", subcore_axis_name="s", num_cores=NC, num_subcores=NS
    )


def _deg_body(edge_ref, ones_ref, zer_ref, out_ref, idxv, onesv, wb, deg_sh):
    c = lax.axis_index("c")
    s = lax.axis_index("s")
    pltpu.sync_copy(edge_ref.at[c, s], idxv)
    pltpu.sync_copy(ones_ref, onesv)
    pltpu.sync_copy(zer_ref, wb)
    base = s * DSTR
    for k in range(DSTR // 80):
        pltpu.sync_copy(wb, deg_sh.at[pl.ds(base + k * 80, 80)])
    plsc.subcore_barrier()

    def chunk(ci, carry):
        pltpu.sync_copy(onesv, deg_sh.at[idxv.at[ci]], add=True)
        return carry

    lax.fori_loop(0, ANCH, chunk, 0)
    plsc.subcore_barrier()

    for k in range(DSTR // 80):
        pltpu.sync_copy(deg_sh.at[pl.ds(base + k * 80, 80)], wb)
        pltpu.sync_copy(wb, out_ref.at[c, pl.ds(base + k * 80, 80)])


_deg_call = pl.kernel(
    _deg_body,
    out_type=jax.ShapeDtypeStruct((NC, NPAD, DW), jnp.float32),
    mesh=_sc_mesh(),
    scratch_types=[
        pltpu.VMEM((ANCH, ACH), jnp.int32),
        pltpu.VMEM((ACH, DW), jnp.float32),
        pltpu.VMEM((80, DW), jnp.float32),
        pltpu.VMEM_SHARED((NPAD, DW), jnp.float32),
    ],
)


def _agg_body(feat_ref, srcr_ref, dstr_ref, z_ref, out_ref, srcv, dstv, rows,
              agg_sh, sem):
    c = lax.axis_index("c")
    s = lax.axis_index("s")
    pltpu.sync_copy(srcr_ref.at[c, s], srcv)
    pltpu.sync_copy(dstr_ref.at[s], dstv)
    pltpu.sync_copy(z_ref, rows)
    base = s * DSTR
    for k in range(DSTR // 128):
        pltpu.sync_copy(rows, agg_sh.at[pl.ds(base + k * 128, 128)])
    plsc.subcore_barrier()

    def chunk(ci, carry):
        pltpu.async_copy(feat_ref.at[srcv.at[ci]], rows, sem).wait()
        pltpu.sync_copy(rows, agg_sh.at[dstv.at[ci]], add=True)
        return carry

    lax.fori_loop(0, GNCH, chunk, 0)
    plsc.subcore_barrier()

    for k in range(DSTR // 128):
        pltpu.sync_copy(agg_sh.at[pl.ds(base + k * 128, 128)], rows)
        pltpu.sync_copy(rows, out_ref.at[c, pl.ds(base + k * 128, 128)])


_agg_call = pl.kernel(
    _agg_body,
    out_type=jax.ShapeDtypeStruct((NC, NPAD, HALF), jnp.float32),
    mesh=_sc_mesh(),
    scratch_types=[
        pltpu.VMEM((GNCH, GCH), jnp.int32),
        pltpu.VMEM((GNCH, GCH), jnp.int32),
        pltpu.VMEM((128, HALF), jnp.float32),
        pltpu.VMEM_SHARED((NPAD, HALF), jnp.float32),
        pltpu.SemaphoreType.DMA,
    ],
)


def _mm1_body(h_ref, w_ref, dout_ref, o_ref):
    nsrc = lax.rsqrt(jnp.maximum(dout_ref[...], 1.0))
    y = jnp.dot(h_ref[...], w_ref[...], preferred_element_type=jnp.float32) * nsrc
    o_ref[0, :, :] = y[:, :HALF]
    o_ref[1, :, :] = y[:, HALF:]


def _mid_body(a_ref, din_ref, dout_ref, b_ref, w_ref, o_ref):
    x = jnp.concatenate([a_ref[0, :, :], a_ref[1, :, :]], axis=1)
    ndst = lax.rsqrt(jnp.maximum(din_ref[...], 1.0))
    h1 = jnp.maximum(x * ndst + b_ref[...], 0.0)
    nsrc = lax.rsqrt(jnp.maximum(dout_ref[...], 1.0))
    y = jnp.dot(h1, w_ref[...], preferred_element_type=jnp.float32) * nsrc
    o_ref[0, :, :] = y[:, :HALF]
    o_ref[1, :, :] = y[:, HALF:]


def _fin_body(a_ref, din_ref, b_ref, z_ref, wl1_ref, bl1_ref, w2r_ref, bl2_ref,
              o_ref, acc_ref):
    i = pl.program_id(0)
    x = jnp.concatenate([a_ref[0, :, :], a_ref[1, :, :]], axis=1)
    ndst = lax.rsqrt(jnp.maximum(din_ref[...], 1.0))
    h2 = jnp.maximum(x * ndst + b_ref[...], 0.0)
    ps = jnp.sum(h2, axis=0, keepdims=True)

    @pl.when(i == 0)
    def _init():
        acc_ref[...] = ps

    @pl.when(i > 0)
    def _acc():
        acc_ref[...] = acc_ref[...] + ps

    @pl.when(i == GRID - 1)
    def _final():
        pooled = acc_ref[...] * (1.0 / NN)
        zz = jnp.dot(z_ref[...], wl1_ref[...],
                     preferred_element_type=jnp.float32) + bl1_ref[...]
        val = (jnp.sum(pooled * w2r_ref[0:1, :])
               + jnp.sum(zz * w2r_ref[1:2, :]) + bl2_ref[0, 0])
        o_ref[...] = val.reshape(1, 1)


def _mm1(h, W1, dout_col):
    return pl.pallas_call(
        _mm1_body,
        grid=(GRID,),
        in_specs=[
            pl.BlockSpec((BM, DD), lambda i: (i, 0)),
            pl.BlockSpec((DD, DD), lambda i: (0, 0)),
            pl.BlockSpec((BM, 1), lambda i: (i, 0)),
        ],
        out_specs=pl.BlockSpec((NC, BM, HALF), lambda i: (0, i, 0)),
        out_shape=jax.ShapeDtypeStruct((NC, NN, HALF), jnp.float32),
    )(h, W1, dout_col)


def _mid(agg, din_col, dout_col, b, W2):
    return pl.pallas_call(
        _mid_body,
        grid=(GRID,),
        in_specs=[
            pl.BlockSpec((NC, BM, HALF), lambda i: (0, i, 0)),
            pl.BlockSpec((BM, 1), lambda i: (i, 0)),
            pl.BlockSpec((BM, 1), lambda i: (i, 0)),
            pl.BlockSpec((1, DD), lambda i: (0, 0)),
            pl.BlockSpec((DD, DD), lambda i: (0, 0)),
        ],
        out_specs=pl.BlockSpec((NC, BM, HALF), lambda i: (0, i, 0)),
        out_shape=jax.ShapeDtypeStruct((NC, NN, HALF), jnp.float32),
    )(agg, din_col, dout_col, b, W2)


def _fin(agg, din_col, b, z, Wl1, bl1, w2r, bl2):
    return pl.pallas_call(
        _fin_body,
        grid=(GRID,),
        in_specs=[
            pl.BlockSpec((NC, BM, HALF), lambda i: (0, i, 0)),
            pl.BlockSpec((BM, 1), lambda i: (i, 0)),
            pl.BlockSpec((1, DD), lambda i: (0, 0)),
            pl.BlockSpec((1, DD), lambda i: (0, 0)),
            pl.BlockSpec((DD, DD), lambda i: (0, 0)),
            pl.BlockSpec((1, DD), lambda i: (0, 0)),
            pl.BlockSpec((2, DD), lambda i: (0, 0)),
            pl.BlockSpec((1, 1), lambda i: (0, 0)),
        ],
        out_specs=pl.BlockSpec((1, 1), lambda i: (0, 0)),
        out_shape=jax.ShapeDtypeStruct((1, 1), jnp.float32),
        scratch_shapes=[pltpu.VMEM((1, DD), jnp.float32)],
    )(agg, din_col, b, z, Wl1, bl1, w2r, bl2)


def kernel(h, edge_index, z, W1, b1, W2, b2, Wl1, bl1, Wl2, bl2):
    src = edge_index[0]
    dst = edge_index[1]
    edge_r = edge_index.reshape(NC, NS, ANCH, ACH)
    padw = ((0, 0), (0, GCH * GNCH - EPT))
    src0 = jnp.pad(src.reshape(NS, EPT), padw)
    src_off = jnp.stack([src0, src0 + NN]).reshape(NC, NS, GNCH, GCH)
    dst_r = jnp.pad(dst.reshape(NS, EPT), padw,
                    constant_values=NN).reshape(NS, GNCH, GCH)
    ones_h = jnp.ones((ACH, DW), jnp.float32)
    zdeg_h = jnp.zeros((80, DW), jnp.float32)
    zrow_h = jnp.zeros((128, HALF), jnp.float32)

    degx = _deg_call(edge_r, ones_h, zdeg_h)
    dout_col = degx[0, :NN, 0:1]
    din_col = degx[1, :NN, 0:1]

    feat1 = _mm1(h, W1, dout_col)
    agg1 = _agg_call(feat1.reshape(NC * NN, HALF), src_off, dst_r, zrow_h)
    feat2 = _mid(agg1, din_col, dout_col, b1.reshape(1, DD), W2)
    agg2 = _agg_call(feat2.reshape(NC * NN, HALF), src_off, dst_r, zrow_h)
    score = _fin(agg2, din_col, b2.reshape(1, DD), z, Wl1,
                 bl1.reshape(1, DD), Wl2.reshape(2, DD), bl2.reshape(1, 1))
    return score


# speedup vs baseline: 1.6607x; 1.6607x over previous
import jax
import jax.numpy as jnp
from jax import lax
from jax.experimental import pallas as pl
from jax.experimental.pallas import tpu as pltpu
from jax.experimental.pallas import tpu_sc as plsc

NN = 10000
EE = 160000
DD = 256
HALF = 128
NC = 2
NS = 16
EPT = EE // NS
ACH = 80
ANCH = 125
DW = 128
NPAD = 10240
DSTR = NPAD // NS
BM = 2000
GRID = NN // BM


def _sc_mesh():
    return plsc.VectorSubcoreMesh(
        core_axis_name="c", subcore_axis_name="s", num_cores=NC, num_subcores=NS
    )


def _deg_body(edge_ref, ones_ref, zer_ref, out_ref, idxv, onesv, wb, deg_sh):
    c = lax.axis_index("c")
    s = lax.axis_index("s")
    pltpu.sync_copy(edge_ref.at[c, s], idxv)
    pltpu.sync_copy(ones_ref, onesv)
    pltpu.sync_copy(zer_ref, wb)
    base = s * DSTR
    for k in range(DSTR // 80):
        pltpu.sync_copy(wb, deg_sh.at[pl.ds(base + k * 80, 80)])
    plsc.subcore_barrier()

    def chunk(ci, carry):
        pltpu.sync_copy(onesv, deg_sh.at[idxv.at[ci]], add=True)
        return carry

    lax.fori_loop(0, ANCH, chunk, 0)
    plsc.subcore_barrier()

    for k in range(DSTR // 80):
        pltpu.sync_copy(deg_sh.at[pl.ds(base + k * 80, 80)], wb)
        pltpu.sync_copy(wb, out_ref.at[c, pl.ds(base + k * 80, 80)])


_deg_call = pl.kernel(
    _deg_body,
    out_type=jax.ShapeDtypeStruct((NC, NPAD, DW), jnp.float32),
    mesh=_sc_mesh(),
    scratch_types=[
        pltpu.VMEM((ANCH, ACH), jnp.int32),
        pltpu.VMEM((ACH, DW), jnp.float32),
        pltpu.VMEM((80, DW), jnp.float32),
        pltpu.VMEM_SHARED((NPAD, DW), jnp.float32),
    ],
)


def _agg_body(feat_ref, srcr_ref, dstr_ref, z_ref, out_ref, srcv, dstv, rows,
              agg_sh, sem, sem2):
    c = lax.axis_index("c")
    s = lax.axis_index("s")
    pltpu.sync_copy(srcr_ref.at[c, s], srcv)
    pltpu.sync_copy(dstr_ref.at[s], dstv)
    zb = rows.at[pl.ds(0, 128)]
    pltpu.sync_copy(z_ref, zb)
    base = s * DSTR
    for k in range(DSTR // 128):
        pltpu.sync_copy(zb, agg_sh.at[pl.ds(base + k * 128, 128)])
    plsc.subcore_barrier()

    ra = rows.at[pl.ds(0, ACH)]
    rb = rows.at[pl.ds(ACH, ACH)]

    def pair(g, carry):
        c0 = 2 * g
        d0 = pltpu.async_copy(
            feat_ref.at[srcv.at[pl.ds(c0 * ACH, ACH)]], ra, sem)
        d1 = pltpu.async_copy(
            feat_ref.at[srcv.at[pl.ds(c0 * ACH + ACH, ACH)]], rb, sem2)
        d0.wait()
        d1.wait()
        pltpu.sync_copy(ra, agg_sh.at[dstv.at[c0]], add=True)
        pltpu.sync_copy(rb, agg_sh.at[dstv.at[c0 + 1]], add=True)
        return carry

    lax.fori_loop(0, ANCH // 2, pair, 0)
    pltpu.async_copy(
        feat_ref.at[srcv.at[pl.ds((ANCH - 1) * ACH, ACH)]], ra, sem).wait()
    pltpu.sync_copy(ra, agg_sh.at[dstv.at[ANCH - 1]], add=True)
    plsc.subcore_barrier()

    for k in range(DSTR // 128):
        pltpu.sync_copy(agg_sh.at[pl.ds(base + k * 128, 128)], zb)
        pltpu.sync_copy(zb, out_ref.at[c, pl.ds(base + k * 128, 128)])


_agg_call = pl.kernel(
    _agg_body,
    out_type=jax.ShapeDtypeStruct((NC, NPAD, HALF), jnp.float32),
    mesh=_sc_mesh(),
    scratch_types=[
        pltpu.VMEM((EPT,), jnp.int32),
        pltpu.VMEM((ANCH, ACH), jnp.int32),
        pltpu.VMEM((2 * ACH, HALF), jnp.float32),
        pltpu.VMEM_SHARED((NPAD, HALF), jnp.float32),
        pltpu.SemaphoreType.DMA,
        pltpu.SemaphoreType.DMA,
    ],
)


def _mm1_body(h_ref, w_ref, dout_ref, o_ref):
    nsrc = lax.rsqrt(jnp.maximum(dout_ref[...], 1.0))
    y = jnp.dot(h_ref[...], w_ref[...], preferred_element_type=jnp.float32) * nsrc
    o_ref[0, :, :] = y[:, :HALF]
    o_ref[1, :, :] = y[:, HALF:]


def _mid_body(a_ref, din_ref, dout_ref, b_ref, w_ref, o_ref):
    x = jnp.concatenate([a_ref[0, :, :], a_ref[1, :, :]], axis=1)
    ndst = lax.rsqrt(jnp.maximum(din_ref[...], 1.0))
    h1 = jnp.maximum(x * ndst + b_ref[...], 0.0)
    nsrc = lax.rsqrt(jnp.maximum(dout_ref[...], 1.0))
    y = jnp.dot(h1, w_ref[...], preferred_element_type=jnp.float32) * nsrc
    o_ref[0, :, :] = y[:, :HALF]
    o_ref[1, :, :] = y[:, HALF:]


def _fin_body(a_ref, din_ref, b_ref, z_ref, wl1_ref, bl1_ref, w2r_ref, bl2_ref,
              o_ref, acc_ref):
    i = pl.program_id(0)
    x = jnp.concatenate([a_ref[0, :, :], a_ref[1, :, :]], axis=1)
    ndst = lax.rsqrt(jnp.maximum(din_ref[...], 1.0))
    h2 = jnp.maximum(x * ndst + b_ref[...], 0.0)
    ps = jnp.sum(h2, axis=0, keepdims=True)

    @pl.when(i == 0)
    def _init():
        acc_ref[...] = ps

    @pl.when(i > 0)
    def _acc():
        acc_ref[...] = acc_ref[...] + ps

    @pl.when(i == GRID - 1)
    def _final():
        pooled = acc_ref[...] * (1.0 / NN)
        zz = jnp.dot(z_ref[...], wl1_ref[...],
                     preferred_element_type=jnp.float32) + bl1_ref[...]
        val = (jnp.sum(pooled * w2r_ref[0:1, :])
               + jnp.sum(zz * w2r_ref[1:2, :]) + bl2_ref[0, 0])
        o_ref[...] = val.reshape(1, 1)


def _mm1(h, W1, dout_col):
    return pl.pallas_call(
        _mm1_body,
        grid=(GRID,),
        in_specs=[
            pl.BlockSpec((BM, DD), lambda i: (i, 0)),
            pl.BlockSpec((DD, DD), lambda i: (0, 0)),
            pl.BlockSpec((BM, 1), lambda i: (i, 0)),
        ],
        out_specs=pl.BlockSpec((NC, BM, HALF), lambda i: (0, i, 0)),
        out_shape=jax.ShapeDtypeStruct((NC, NN, HALF), jnp.float32),
    )(h, W1, dout_col)


def _mid(agg, din_col, dout_col, b, W2):
    return pl.pallas_call(
        _mid_body,
        grid=(GRID,),
        in_specs=[
            pl.BlockSpec((NC, BM, HALF), lambda i: (0, i, 0)),
            pl.BlockSpec((BM, 1), lambda i: (i, 0)),
            pl.BlockSpec((BM, 1), lambda i: (i, 0)),
            pl.BlockSpec((1, DD), lambda i: (0, 0)),
            pl.BlockSpec((DD, DD), lambda i: (0, 0)),
        ],
        out_specs=pl.BlockSpec((NC, BM, HALF), lambda i: (0, i, 0)),
        out_shape=jax.ShapeDtypeStruct((NC, NN, HALF), jnp.float32),
    )(agg, din_col, dout_col, b, W2)


def _fin(agg, din_col, b, z, Wl1, bl1, w2r, bl2):
    return pl.pallas_call(
        _fin_body,
        grid=(GRID,),
        in_specs=[
            pl.BlockSpec((NC, BM, HALF), lambda i: (0, i, 0)),
            pl.BlockSpec((BM, 1), lambda i: (i, 0)),
            pl.BlockSpec((1, DD), lambda i: (0, 0)),
            pl.BlockSpec((1, DD), lambda i: (0, 0)),
            pl.BlockSpec((DD, DD), lambda i: (0, 0)),
            pl.BlockSpec((1, DD), lambda i: (0, 0)),
            pl.BlockSpec((2, DD), lambda i: (0, 0)),
            pl.BlockSpec((1, 1), lambda i: (0, 0)),
        ],
        out_specs=pl.BlockSpec((1, 1), lambda i: (0, 0)),
        out_shape=jax.ShapeDtypeStruct((1, 1), jnp.float32),
        scratch_shapes=[pltpu.VMEM((1, DD), jnp.float32)],
    )(agg, din_col, b, z, Wl1, bl1, w2r, bl2)


def kernel(h, edge_index, z, W1, b1, W2, b2, Wl1, bl1, Wl2, bl2):
    src = edge_index[0]
    dst = edge_index[1]
    edge_r = edge_index.reshape(NC, NS, ANCH, ACH)
    src_off = jnp.stack([src, src + NN]).reshape(NC, NS, EPT)
    dst_r = dst.reshape(NS, ANCH, ACH)
    ones_h = jnp.ones((ACH, DW), jnp.float32)
    zdeg_h = jnp.zeros((80, DW), jnp.float32)
    zrow_h = jnp.zeros((128, HALF), jnp.float32)

    degx = _deg_call(edge_r, ones_h, zdeg_h)
    dout_col = degx[0, :NN, 0:1]
    din_col = degx[1, :NN, 0:1]

    feat1 = _mm1(h, W1, dout_col)
    agg1 = _agg_call(feat1.reshape(NC * NN, HALF), src_off, dst_r, zrow_h)
    feat2 = _mid(agg1, din_col, dout_col, b1.reshape(1, DD), W2)
    agg2 = _agg_call(feat2.reshape(NC * NN, HALF), src_off, dst_r, zrow_h)
    score = _fin(agg2, din_col, b2.reshape(1, DD), z, Wl1,
                 bl1.reshape(1, DD), Wl2.reshape(2, DD), bl2.reshape(1, 1))
    return score


# speedup vs baseline: 1.7173x; 1.0341x over previous
import jax
import jax.numpy as jnp
from jax import lax
from jax.experimental import pallas as pl
from jax.experimental.pallas import tpu as pltpu
from jax.experimental.pallas import tpu_sc as plsc

NN = 10000
EE = 160000
DD = 256
HALF = 128
NC = 2
NS = 16
EPT = EE // NS
ACH = 80
ANCH = 125
DW = 128
NPAD = 10240
DSTR = NPAD // NS
BM = 2000
GRID = NN // BM


def _sc_mesh():
    return plsc.VectorSubcoreMesh(
        core_axis_name="c", subcore_axis_name="s", num_cores=NC, num_subcores=NS
    )


def _deg_body(edge_ref, ones_ref, zer_ref, out_ref, idxv, onesv, wb, deg_sh):
    c = lax.axis_index("c")
    s = lax.axis_index("s")
    pltpu.sync_copy(edge_ref.at[c, s], idxv)
    pltpu.sync_copy(ones_ref, onesv)
    pltpu.sync_copy(zer_ref, wb)
    base = s * DSTR
    for k in range(DSTR // 80):
        pltpu.sync_copy(wb, deg_sh.at[pl.ds(base + k * 80, 80)])
    plsc.subcore_barrier()

    def chunk(ci, carry):
        pltpu.sync_copy(onesv, deg_sh.at[idxv.at[ci]], add=True)
        return carry

    lax.fori_loop(0, ANCH, chunk, 0)
    plsc.subcore_barrier()

    for k in range(DSTR // 80):
        pltpu.sync_copy(deg_sh.at[pl.ds(base + k * 80, 80)], wb)
        pltpu.sync_copy(wb, out_ref.at[c, pl.ds(base + k * 80, 80)])


_deg_call = pl.kernel(
    _deg_body,
    out_type=jax.ShapeDtypeStruct((NC, NPAD, DW), jnp.float32),
    mesh=_sc_mesh(),
    scratch_types=[
        pltpu.VMEM((ANCH, ACH), jnp.int32),
        pltpu.VMEM((ACH, DW), jnp.float32),
        pltpu.VMEM((80, DW), jnp.float32),
        pltpu.VMEM_SHARED((NPAD, DW), jnp.float32),
    ],
)


def _agg_body(feat_ref, srcr_ref, dstr_ref, z_ref, out_ref, srcv, dstv, rows,
              agg_sh, sem, sem2):
    c = lax.axis_index("c")
    s = lax.axis_index("s")
    pltpu.sync_copy(srcr_ref.at[c, s], srcv)
    pltpu.sync_copy(dstr_ref.at[s], dstv)
    zb = rows.at[pl.ds(0, 128)]
    pltpu.sync_copy(z_ref, zb)
    base = s * DSTR
    for k in range(DSTR // 128):
        pltpu.sync_copy(zb, agg_sh.at[pl.ds(base + k * 128, 128)])
    plsc.subcore_barrier()

    ra = rows.at[pl.ds(0, ACH)]
    rb = rows.at[pl.ds(ACH, ACH)]

    pltpu.async_copy(feat_ref.at[srcv.at[pl.ds(0, ACH)]], ra, sem)

    def pair(g, carry):
        c0 = 2 * g
        pltpu.make_async_copy(
            feat_ref.at[srcv.at[pl.ds(c0 * ACH, ACH)]], ra, sem).wait()
        pltpu.async_copy(
            feat_ref.at[srcv.at[pl.ds((c0 + 1) * ACH, ACH)]], rb, sem2)
        pltpu.sync_copy(ra, agg_sh.at[dstv.at[c0]], add=True)
        pltpu.make_async_copy(
            feat_ref.at[srcv.at[pl.ds((c0 + 1) * ACH, ACH)]], rb, sem2).wait()
        pltpu.async_copy(
            feat_ref.at[srcv.at[pl.ds((c0 + 2) * ACH, ACH)]], ra, sem)
        pltpu.sync_copy(rb, agg_sh.at[dstv.at[c0 + 1]], add=True)
        return carry

    lax.fori_loop(0, ANCH // 2, pair, 0)
    pltpu.make_async_copy(
        feat_ref.at[srcv.at[pl.ds((ANCH - 1) * ACH, ACH)]], ra, sem).wait()
    pltpu.sync_copy(ra, agg_sh.at[dstv.at[ANCH - 1]], add=True)
    plsc.subcore_barrier()

    for k in range(DSTR // 128):
        pltpu.sync_copy(agg_sh.at[pl.ds(base + k * 128, 128)], zb)
        pltpu.sync_copy(zb, out_ref.at[c, pl.ds(base + k * 128, 128)])


_agg_call = pl.kernel(
    _agg_body,
    out_type=jax.ShapeDtypeStruct((NC, NPAD, HALF), jnp.float32),
    mesh=_sc_mesh(),
    scratch_types=[
        pltpu.VMEM((EPT,), jnp.int32),
        pltpu.VMEM((ANCH, ACH), jnp.int32),
        pltpu.VMEM((2 * ACH, HALF), jnp.float32),
        pltpu.VMEM_SHARED((NPAD, HALF), jnp.float32),
        pltpu.SemaphoreType.DMA,
        pltpu.SemaphoreType.DMA,
    ],
)


def _mm1_body(h_ref, w_ref, dout_ref, o_ref):
    nsrc = lax.rsqrt(jnp.maximum(dout_ref[...], 1.0))
    y = jnp.dot(h_ref[...], w_ref[...], preferred_element_type=jnp.float32) * nsrc
    o_ref[0, :, :] = y[:, :HALF]
    o_ref[1, :, :] = y[:, HALF:]


def _mid_body(a_ref, din_ref, dout_ref, b_ref, w_ref, o_ref):
    x = jnp.concatenate([a_ref[0, :, :], a_ref[1, :, :]], axis=1)
    ndst = lax.rsqrt(jnp.maximum(din_ref[...], 1.0))
    h1 = jnp.maximum(x * ndst + b_ref[...], 0.0)
    nsrc = lax.rsqrt(jnp.maximum(dout_ref[...], 1.0))
    y = jnp.dot(h1, w_ref[...], preferred_element_type=jnp.float32) * nsrc
    o_ref[0, :, :] = y[:, :HALF]
    o_ref[1, :, :] = y[:, HALF:]


def _fin_body(a_ref, din_ref, b_ref, z_ref, wl1_ref, bl1_ref, w2r_ref, bl2_ref,
              o_ref, acc_ref):
    i = pl.program_id(0)
    x = jnp.concatenate([a_ref[0, :, :], a_ref[1, :, :]], axis=1)
    ndst = lax.rsqrt(jnp.maximum(din_ref[...], 1.0))
    h2 = jnp.maximum(x * ndst + b_ref[...], 0.0)
    ps = jnp.sum(h2, axis=0, keepdims=True)

    @pl.when(i == 0)
    def _init():
        acc_ref[...] = ps

    @pl.when(i > 0)
    def _acc():
        acc_ref[...] = acc_ref[...] + ps

    @pl.when(i == GRID - 1)
    def _final():
        pooled = acc_ref[...] * (1.0 / NN)
        zz = jnp.dot(z_ref[...], wl1_ref[...],
                     preferred_element_type=jnp.float32) + bl1_ref[...]
        val = (jnp.sum(pooled * w2r_ref[0:1, :])
               + jnp.sum(zz * w2r_ref[1:2, :]) + bl2_ref[0, 0])
        o_ref[...] = val.reshape(1, 1)


def _mm1(h, W1, dout_col):
    return pl.pallas_call(
        _mm1_body,
        grid=(GRID,),
        in_specs=[
            pl.BlockSpec((BM, DD), lambda i: (i, 0)),
            pl.BlockSpec((DD, DD), lambda i: (0, 0)),
            pl.BlockSpec((BM, 1), lambda i: (i, 0)),
        ],
        out_specs=pl.BlockSpec((NC, BM, HALF), lambda i: (0, i, 0)),
        out_shape=jax.ShapeDtypeStruct((NC, NN, HALF), jnp.float32),
    )(h, W1, dout_col)


def _mid(agg, din_col, dout_col, b, W2):
    return pl.pallas_call(
        _mid_body,
        grid=(GRID,),
        in_specs=[
            pl.BlockSpec((NC, BM, HALF), lambda i: (0, i, 0)),
            pl.BlockSpec((BM, 1), lambda i: (i, 0)),
            pl.BlockSpec((BM, 1), lambda i: (i, 0)),
            pl.BlockSpec((1, DD), lambda i: (0, 0)),
            pl.BlockSpec((DD, DD), lambda i: (0, 0)),
        ],
        out_specs=pl.BlockSpec((NC, BM, HALF), lambda i: (0, i, 0)),
        out_shape=jax.ShapeDtypeStruct((NC, NN, HALF), jnp.float32),
    )(agg, din_col, dout_col, b, W2)


def _fin(agg, din_col, b, z, Wl1, bl1, w2r, bl2):
    return pl.pallas_call(
        _fin_body,
        grid=(GRID,),
        in_specs=[
            pl.BlockSpec((NC, BM, HALF), lambda i: (0, i, 0)),
            pl.BlockSpec((BM, 1), lambda i: (i, 0)),
            pl.BlockSpec((1, DD), lambda i: (0, 0)),
            pl.BlockSpec((1, DD), lambda i: (0, 0)),
            pl.BlockSpec((DD, DD), lambda i: (0, 0)),
            pl.BlockSpec((1, DD), lambda i: (0, 0)),
            pl.BlockSpec((2, DD), lambda i: (0, 0)),
            pl.BlockSpec((1, 1), lambda i: (0, 0)),
        ],
        out_specs=pl.BlockSpec((1, 1), lambda i: (0, 0)),
        out_shape=jax.ShapeDtypeStruct((1, 1), jnp.float32),
        scratch_shapes=[pltpu.VMEM((1, DD), jnp.float32)],
    )(agg, din_col, b, z, Wl1, bl1, w2r, bl2)


def kernel(h, edge_index, z, W1, b1, W2, b2, Wl1, bl1, Wl2, bl2):
    src = edge_index[0]
    dst = edge_index[1]
    edge_r = edge_index.reshape(NC, NS, ANCH, ACH)
    src_off = jnp.stack([src, src + NN]).reshape(NC, NS, EPT)
    dst_r = dst.reshape(NS, ANCH, ACH)
    ones_h = jnp.ones((ACH, DW), jnp.float32)
    zdeg_h = jnp.zeros((80, DW), jnp.float32)
    zrow_h = jnp.zeros((128, HALF), jnp.float32)

    degx = _deg_call(edge_r, ones_h, zdeg_h)
    dout_col = degx[0, :NN, 0:1]
    din_col = degx[1, :NN, 0:1]

    feat1 = _mm1(h, W1, dout_col)
    agg1 = _agg_call(feat1.reshape(NC * NN, HALF), src_off, dst_r, zrow_h)
    feat2 = _mid(agg1, din_col, dout_col, b1.reshape(1, DD), W2)
    agg2 = _agg_call(feat2.reshape(NC * NN, HALF), src_off, dst_r, zrow_h)
    score = _fin(agg2, din_col, b2.reshape(1, DD), z, Wl1,
                 bl1.reshape(1, DD), Wl2.reshape(2, DD), bl2.reshape(1, 1))
    return score


# speedup vs baseline: 1.7206x; 1.0019x over previous
import jax
import jax.numpy as jnp
from jax import lax
from jax.experimental import pallas as pl
from jax.experimental.pallas import tpu as pltpu
from jax.experimental.pallas import tpu_sc as plsc

NN = 10000
EE = 160000
DD = 256
HALF = 128
NC = 2
NS = 16
EPT = EE // NS
ACH = 80
ANCH = 125
DW = 128
NPAD = 10240
DSTR = NPAD // NS
BM = 2000
GRID = NN // BM


def _sc_mesh():
    return plsc.VectorSubcoreMesh(
        core_axis_name="c", subcore_axis_name="s", num_cores=NC, num_subcores=NS
    )


def _deg_body(edge_ref, ones_ref, zer_ref, out_ref, idxv, onesv, wb, deg_sh,
              sem, sem2):
    c = lax.axis_index("c")
    s = lax.axis_index("s")
    pltpu.sync_copy(edge_ref.at[c, s], idxv)
    pltpu.sync_copy(ones_ref, onesv)
    pltpu.sync_copy(zer_ref, wb)
    base = s * DSTR
    for k in range(DSTR // 80):
        pltpu.sync_copy(wb, deg_sh.at[pl.ds(base + k * 80, 80)])
    plsc.subcore_barrier()

    def pair(g, carry):
        c0 = 2 * g
        d0 = pltpu.async_copy(onesv, deg_sh.at[idxv.at[c0]], sem, add=True)
        d1 = pltpu.async_copy(onesv, deg_sh.at[idxv.at[c0 + 1]], sem2, add=True)
        d0.wait()
        d1.wait()
        return carry

    lax.fori_loop(0, ANCH // 2, pair, 0)
    pltpu.async_copy(onesv, deg_sh.at[idxv.at[ANCH - 1]], sem, add=True).wait()
    plsc.subcore_barrier()

    for k in range(DSTR // 80):
        pltpu.sync_copy(deg_sh.at[pl.ds(base + k * 80, 80)], wb)
        pltpu.sync_copy(wb, out_ref.at[c, pl.ds(base + k * 80, 80)])


_deg_call = pl.kernel(
    _deg_body,
    out_type=jax.ShapeDtypeStruct((NC, NPAD, DW), jnp.float32),
    mesh=_sc_mesh(),
    scratch_types=[
        pltpu.VMEM((ANCH, ACH), jnp.int32),
        pltpu.VMEM((ACH, DW), jnp.float32),
        pltpu.VMEM((80, DW), jnp.float32),
        pltpu.VMEM_SHARED((NPAD, DW), jnp.float32),
        pltpu.SemaphoreType.DMA,
        pltpu.SemaphoreType.DMA,
    ],
)


def _agg_body(feat_ref, srcr_ref, dstr_ref, z_ref, out_ref, srcv, dstv, rows,
              agg_sh, sem, sem2):
    c = lax.axis_index("c")
    s = lax.axis_index("s")
    pltpu.sync_copy(srcr_ref.at[c, s], srcv)
    pltpu.sync_copy(dstr_ref.at[s], dstv)
    zb = rows.at[pl.ds(0, 128)]
    pltpu.sync_copy(z_ref, zb)
    base = s * DSTR
    for k in range(DSTR // 128):
        pltpu.sync_copy(zb, agg_sh.at[pl.ds(base + k * 128, 128)])
    plsc.subcore_barrier()

    ra = rows.at[pl.ds(0, ACH)]
    rb = rows.at[pl.ds(ACH, ACH)]

    pltpu.async_copy(feat_ref.at[srcv.at[pl.ds(0, ACH)]], ra, sem)

    def pair(g, carry):
        c0 = 2 * g
        pltpu.make_async_copy(
            feat_ref.at[srcv.at[pl.ds(c0 * ACH, ACH)]], ra, sem).wait()
        pltpu.async_copy(
            feat_ref.at[srcv.at[pl.ds((c0 + 1) * ACH, ACH)]], rb, sem2)
        pltpu.sync_copy(ra, agg_sh.at[dstv.at[c0]], add=True)
        pltpu.make_async_copy(
            feat_ref.at[srcv.at[pl.ds((c0 + 1) * ACH, ACH)]], rb, sem2).wait()
        pltpu.async_copy(
            feat_ref.at[srcv.at[pl.ds((c0 + 2) * ACH, ACH)]], ra, sem)
        pltpu.sync_copy(rb, agg_sh.at[dstv.at[c0 + 1]], add=True)
        return carry

    lax.fori_loop(0, ANCH // 2, pair, 0)
    pltpu.make_async_copy(
        feat_ref.at[srcv.at[pl.ds((ANCH - 1) * ACH, ACH)]], ra, sem).wait()
    pltpu.sync_copy(ra, agg_sh.at[dstv.at[ANCH - 1]], add=True)
    plsc.subcore_barrier()

    for k in range(DSTR // 128):
        pltpu.sync_copy(agg_sh.at[pl.ds(base + k * 128, 128)], zb)
        pltpu.sync_copy(zb, out_ref.at[c, pl.ds(base + k * 128, 128)])


_agg_call = pl.kernel(
    _agg_body,
    out_type=jax.ShapeDtypeStruct((NC, NPAD, HALF), jnp.float32),
    mesh=_sc_mesh(),
    scratch_types=[
        pltpu.VMEM((EPT,), jnp.int32),
        pltpu.VMEM((ANCH, ACH), jnp.int32),
        pltpu.VMEM((2 * ACH, HALF), jnp.float32),
        pltpu.VMEM_SHARED((NPAD, HALF), jnp.float32),
        pltpu.SemaphoreType.DMA,
        pltpu.SemaphoreType.DMA,
    ],
)


def _mm1_body(h_ref, w_ref, dout_ref, o_ref):
    nsrc = lax.rsqrt(jnp.maximum(dout_ref[...], 1.0))
    y = jnp.dot(h_ref[...], w_ref[...], preferred_element_type=jnp.float32) * nsrc
    o_ref[0, :, :] = y[:, :HALF]
    o_ref[1, :, :] = y[:, HALF:]


def _mid_body(a_ref, din_ref, dout_ref, b_ref, w_ref, o_ref):
    x = jnp.concatenate([a_ref[0, :, :], a_ref[1, :, :]], axis=1)
    ndst = lax.rsqrt(jnp.maximum(din_ref[...], 1.0))
    h1 = jnp.maximum(x * ndst + b_ref[...], 0.0)
    nsrc = lax.rsqrt(jnp.maximum(dout_ref[...], 1.0))
    y = jnp.dot(h1, w_ref[...], preferred_element_type=jnp.float32) * nsrc
    o_ref[0, :, :] = y[:, :HALF]
    o_ref[1, :, :] = y[:, HALF:]


def _fin_body(a_ref, din_ref, b_ref, z_ref, wl1_ref, bl1_ref, w2r_ref, bl2_ref,
              o_ref, acc_ref):
    i = pl.program_id(0)
    x = jnp.concatenate([a_ref[0, :, :], a_ref[1, :, :]], axis=1)
    ndst = lax.rsqrt(jnp.maximum(din_ref[...], 1.0))
    h2 = jnp.maximum(x * ndst + b_ref[...], 0.0)
    ps = jnp.sum(h2, axis=0, keepdims=True)

    @pl.when(i == 0)
    def _init():
        acc_ref[...] = ps

    @pl.when(i > 0)
    def _acc():
        acc_ref[...] = acc_ref[...] + ps

    @pl.when(i == GRID - 1)
    def _final():
        pooled = acc_ref[...] * (1.0 / NN)
        zz = jnp.dot(z_ref[...], wl1_ref[...],
                     preferred_element_type=jnp.float32) + bl1_ref[...]
        val = (jnp.sum(pooled * w2r_ref[0:1, :])
               + jnp.sum(zz * w2r_ref[1:2, :]) + bl2_ref[0, 0])
        o_ref[...] = val.reshape(1, 1)


def _mm1(h, W1, dout_col):
    return pl.pallas_call(
        _mm1_body,
        grid=(GRID,),
        in_specs=[
            pl.BlockSpec((BM, DD), lambda i: (i, 0)),
            pl.BlockSpec((DD, DD), lambda i: (0, 0)),
            pl.BlockSpec((BM, 1), lambda i: (i, 0)),
        ],
        out_specs=pl.BlockSpec((NC, BM, HALF), lambda i: (0, i, 0)),
        out_shape=jax.ShapeDtypeStruct((NC, NN, HALF), jnp.float32),
    )(h, W1, dout_col)


def _mid(agg, din_col, dout_col, b, W2):
    return pl.pallas_call(
        _mid_body,
        grid=(GRID,),
        in_specs=[
            pl.BlockSpec((NC, BM, HALF), lambda i: (0, i, 0)),
            pl.BlockSpec((BM, 1), lambda i: (i, 0)),
            pl.BlockSpec((BM, 1), lambda i: (i, 0)),
            pl.BlockSpec((1, DD), lambda i: (0, 0)),
            pl.BlockSpec((DD, DD), lambda i: (0, 0)),
        ],
        out_specs=pl.BlockSpec((NC, BM, HALF), lambda i: (0, i, 0)),
        out_shape=jax.ShapeDtypeStruct((NC, NN, HALF), jnp.float32),
    )(agg, din_col, dout_col, b, W2)


def _fin(agg, din_col, b, z, Wl1, bl1, w2r, bl2):
    return pl.pallas_call(
        _fin_body,
        grid=(GRID,),
        in_specs=[
            pl.BlockSpec((NC, BM, HALF), lambda i: (0, i, 0)),
            pl.BlockSpec((BM, 1), lambda i: (i, 0)),
            pl.BlockSpec((1, DD), lambda i: (0, 0)),
            pl.BlockSpec((1, DD), lambda i: (0, 0)),
            pl.BlockSpec((DD, DD), lambda i: (0, 0)),
            pl.BlockSpec((1, DD), lambda i: (0, 0)),
            pl.BlockSpec((2, DD), lambda i: (0, 0)),
            pl.BlockSpec((1, 1), lambda i: (0, 0)),
        ],
        out_specs=pl.BlockSpec((1, 1), lambda i: (0, 0)),
        out_shape=jax.ShapeDtypeStruct((1, 1), jnp.float32),
        scratch_shapes=[pltpu.VMEM((1, DD), jnp.float32)],
    )(agg, din_col, b, z, Wl1, bl1, w2r, bl2)


def kernel(h, edge_index, z, W1, b1, W2, b2, Wl1, bl1, Wl2, bl2):
    src = edge_index[0]
    dst = edge_index[1]
    edge_r = edge_index.reshape(NC, NS, ANCH, ACH)
    src_off = jnp.stack([src, src + NN]).reshape(NC, NS, EPT)
    dst_r = dst.reshape(NS, ANCH, ACH)
    ones_h = jnp.ones((ACH, DW), jnp.float32)
    zdeg_h = jnp.zeros((80, DW), jnp.float32)
    zrow_h = jnp.zeros((128, HALF), jnp.float32)

    degx = _deg_call(edge_r, ones_h, zdeg_h)
    dout_col = degx[0, :NN, 0:1]
    din_col = degx[1, :NN, 0:1]

    feat1 = _mm1(h, W1, dout_col)
    agg1 = _agg_call(feat1.reshape(NC * NN, HALF), src_off, dst_r, zrow_h)
    feat2 = _mid(agg1, din_col, dout_col, b1.reshape(1, DD), W2)
    agg2 = _agg_call(feat2.reshape(NC * NN, HALF), src_off, dst_r, zrow_h)
    score = _fin(agg2, din_col, b2.reshape(1, DD), z, Wl1,
                 bl1.reshape(1, DD), Wl2.reshape(2, DD), bl2.reshape(1, 1))
    return score
